# baseline (speedup 1.0000x reference)
import functools

import jax
import jax.numpy as jnp
from jax import lax
from jax.experimental import pallas as pl
from jax.experimental.pallas import tpu as pltpu
from jax.experimental.pallas import tpu_sc as plsc

_NC = 2
_NS = 16
_L = 16
_NW = _NC * _NS

_TOTAL = 2 * 2048 * 2048
_ROWS = _TOTAL // _L
_ROWS_PER_W = _ROWS // _NW
_CHUNK_ROWS = 4096


def _make_sc_scale():
    mesh = plsc.VectorSubcoreMesh(core_axis_name="c", subcore_axis_name="s")

    @functools.partial(
        pl.kernel,
        mesh=mesh,
        out_type=jax.ShapeDtypeStruct((_ROWS, _L), jnp.float32),
        scratch_types=[pltpu.VMEM((_CHUNK_ROWS, _L), jnp.float32)],
        compiler_params=pltpu.CompilerParams(use_tc_tiling_on_sc=False),
    )
    def sc_scale(hs_hbm, out_hbm, buf):
        wid = lax.axis_index("s") * _NC + lax.axis_index("c")
        base = wid * _ROWS_PER_W

        def chunk(k, carry):
            off = base + k * _CHUNK_ROWS
            pltpu.sync_copy(hs_hbm.at[pl.ds(off, _CHUNK_ROWS)], buf)

            def row(i, c):
                buf[i] = buf[i] * 1.05
                return c

            lax.fori_loop(0, _CHUNK_ROWS, row, 0)
            pltpu.sync_copy(buf, out_hbm.at[pl.ds(off, _CHUNK_ROWS)])
            return carry

        lax.fori_loop(0, _ROWS_PER_W // _CHUNK_ROWS, chunk, 0)

    return sc_scale


_sc_scale = _make_sc_scale()


@jax.jit
def kernel(hidden_states, gate_weight):
    b, s, h = hidden_states.shape
    hs = hidden_states.reshape(_ROWS, _L)
    out = _sc_scale(hs)
    return out.reshape(b, s, h)


# speedup vs baseline: 7.9442x; 7.9442x over previous
import functools

import jax
import jax.numpy as jnp
from jax.experimental import pallas as pl
from jax.experimental.pallas import tpu as pltpu

_NUM_EXPERTS = 64
_TOP_K = 8
_TB = 512
_NBUF = 3


def _route_scale(hs, gw):
    logits = jax.lax.dot_general(
        hs, gw,
        dimension_numbers=(((1,), (1,)), ((), ())),
        preferred_element_type=jnp.float32,
    )
    cur = logits
    m = None
    kth = None
    for _ in range(_TOP_K):
        kth = jnp.max(cur, axis=-1, keepdims=True)
        if m is None:
            m = kth
        cur = jnp.where(cur >= kth, -jnp.inf, cur)
    sel = logits >= kth
    e = jnp.exp(logits - m)
    q = jnp.sum(jnp.where(sel, e, 0.0), axis=-1, keepdims=True)
    norm_sum = q / q
    return hs * (1.05 * norm_sum)


def _moe_kernel(hs_hbm, gw_ref, out_hbm, in_buf, out_buf, in_sem, out_sem):
    n = hs_hbm.shape[0] // _TB

    def in_cp(i):
        return pltpu.make_async_copy(
            hs_hbm.at[pl.ds(i * _TB, _TB)], in_buf.at[i % _NBUF],
            in_sem.at[i % _NBUF])

    def out_cp(i):
        return pltpu.make_async_copy(
            out_buf.at[i % _NBUF], out_hbm.at[pl.ds(i * _TB, _TB)],
            out_sem.at[i % _NBUF])

    for i in range(_NBUF):
        in_cp(i).start()
    gw = gw_ref[...]
    for i in range(n):
        slot = i % _NBUF
        in_cp(i).wait()
        if i >= _NBUF:
            out_cp(i - _NBUF).wait()
        out_buf[slot] = _route_scale(in_buf[slot], gw)
        out_cp(i).start()
        if i + _NBUF < n:
            in_cp(i + _NBUF).start()
    for i in range(n - _NBUF, n):
        out_cp(i).wait()


@functools.partial(jax.jit, static_argnames=())
def kernel(hidden_states, gate_weight):
    b, s, h = hidden_states.shape
    t = b * s
    hs = hidden_states.reshape(t, h)
    out = pl.pallas_call(
        _moe_kernel,
        in_specs=[
            pl.BlockSpec(memory_space=pl.ANY),
            pl.BlockSpec(memory_space=pltpu.VMEM),
        ],
        out_specs=pl.BlockSpec(memory_space=pl.ANY),
        out_shape=jax.ShapeDtypeStruct((t, h), hidden_states.dtype),
        scratch_shapes=[
            pltpu.VMEM((_NBUF, _TB, h), jnp.float32),
            pltpu.VMEM((_NBUF, _TB, h), jnp.float32),
            pltpu.SemaphoreType.DMA((_NBUF,)),
            pltpu.SemaphoreType.DMA((_NBUF,)),
        ],
    )(hs, gate_weight)
    return out.reshape(b, s, h)


# speedup vs baseline: 8.5546x; 1.0768x over previous
import functools

import jax
import jax.numpy as jnp
from jax.experimental import pallas as pl
from jax.experimental.pallas import tpu as pltpu

_NUM_EXPERTS = 64
_TOP_K = 8
_TB = 512
_NBUF = 4


def _route_scale(hs, gw):
    logits = jax.lax.dot_general(
        hs, gw,
        dimension_numbers=(((1,), (1,)), ((), ())),
        preferred_element_type=jnp.float32,
    )
    cur = logits
    m = None
    kth = None
    for _ in range(_TOP_K):
        kth = jnp.max(cur, axis=-1, keepdims=True)
        if m is None:
            m = kth
        cur = jnp.where(cur >= kth, -jnp.inf, cur)
    sel = logits >= kth
    e = jnp.exp(logits - m)
    q = jnp.sum(jnp.where(sel, e, 0.0), axis=-1, keepdims=True)
    norm_sum = q / q
    return hs * (1.05 * norm_sum)


def _moe_kernel(hs_hbm, gw_ref, out_hbm, in_buf, out_buf, in_sem, out_sem):
    n = hs_hbm.shape[0] // _TB

    def in_cp(i):
        return pltpu.make_async_copy(
            hs_hbm.at[pl.ds(i * _TB, _TB)], in_buf.at[i % _NBUF],
            in_sem.at[i % _NBUF])

    def out_cp(i):
        return pltpu.make_async_copy(
            out_buf.at[i % _NBUF], out_hbm.at[pl.ds(i * _TB, _TB)],
            out_sem.at[i % _NBUF])

    for i in range(_NBUF):
        in_cp(i).start()
    gw = gw_ref[...]
    for i in range(n):
        slot = i % _NBUF
        in_cp(i).wait()
        if i >= _NBUF:
            out_cp(i - _NBUF).wait()
        out_buf[slot] = _route_scale(in_buf[slot], gw)
        out_cp(i).start()
        if i + _NBUF < n:
            in_cp(i + _NBUF).start()
    for i in range(n - _NBUF, n):
        out_cp(i).wait()


@functools.partial(jax.jit, static_argnames=())
def kernel(hidden_states, gate_weight):
    b, s, h = hidden_states.shape
    t = b * s
    hs = hidden_states.reshape(t, h)
    out = pl.pallas_call(
        _moe_kernel,
        in_specs=[
            pl.BlockSpec(memory_space=pl.ANY),
            pl.BlockSpec(memory_space=pltpu.VMEM),
        ],
        out_specs=pl.BlockSpec(memory_space=pl.ANY),
        out_shape=jax.ShapeDtypeStruct((t, h), hidden_states.dtype),
        scratch_shapes=[
            pltpu.VMEM((_NBUF, _TB, h), jnp.float32),
            pltpu.VMEM((_NBUF, _TB, h), jnp.float32),
            pltpu.SemaphoreType.DMA((_NBUF,)),
            pltpu.SemaphoreType.DMA((_NBUF,)),
        ],
    )(hs, gate_weight)
    return out.reshape(b, s, h)
